# baseline (speedup 1.0000x reference)
import functools

import jax
import jax.numpy as jnp
import numpy as np
from jax.experimental import pallas as pl


BATCH = 16384
NG = 64
DIM = 64
BB = 2048
GC = 8
CHUNK = GC * DIM


GH = 8
GL = 8
WIDE = GH * DIM


def _body(k_ref, eps_ref, w_ref, mu_ref, exp_ref, tile_ref, fold_ref, out_ref):
    f32 = jnp.float32
    eps = eps_ref[...]
    kb = k_ref[...]
    kh = kb >> 3
    kl = kb & 7
    i8 = jax.lax.broadcasted_iota(jnp.int32, (BB, GH), 1)
    ohh = (i8 == kh).astype(f32)
    ohl = (i8 == kl).astype(f32)
    eps_t = jnp.dot(eps, tile_ref[...], preferred_element_type=f32)
    e1 = eps_t * jnp.dot(ohh, exp_ref[...], preferred_element_type=f32)
    y = jnp.dot(e1, w_ref[...], preferred_element_type=f32)
    ym = y * jnp.dot(ohl, exp_ref[...], preferred_element_type=f32)
    yf = jnp.dot(ym, fold_ref[...], preferred_element_type=f32)
    g64 = jax.lax.broadcasted_iota(jnp.int32, (BB, NG), 1)
    oh = (g64 == kb).astype(f32)
    out_ref[...] = yf + jnp.dot(oh, mu_ref[...], preferred_element_type=f32)


@jax.jit
def _run(k_col, eps, w, mu, exp_m, tile_m, fold_m):
    grid = (BATCH // BB,)
    return pl.pallas_call(
        _body,
        grid=grid,
        in_specs=[
            pl.BlockSpec((BB, 1), lambda i: (i, 0)),
            pl.BlockSpec((BB, DIM), lambda i: (i, 0)),
            pl.BlockSpec((WIDE, WIDE), lambda i: (0, 0)),
            pl.BlockSpec((NG, DIM), lambda i: (0, 0)),
            pl.BlockSpec((GH, WIDE), lambda i: (0, 0)),
            pl.BlockSpec((DIM, WIDE), lambda i: (0, 0)),
            pl.BlockSpec((WIDE, DIM), lambda i: (0, 0)),
        ],
        out_specs=pl.BlockSpec((BB, DIM), lambda i: (i, 0)),
        out_shape=jax.ShapeDtypeStruct((BATCH, DIM), jnp.float32),
    )(k_col, eps, w, mu, exp_m, tile_m, fold_m)


_c = np.arange(WIDE)
_EXP = (np.arange(GH)[:, None] == (_c[None, :] >> 6)).astype(np.float32)
_TILE = (np.arange(DIM)[:, None] == (_c[None, :] & 63)).astype(np.float32)
_FOLD = _TILE.T.copy()


def kernel(batch_size, k, epsilon, mu, A):
    k_col = k.astype(jnp.int32).reshape(BATCH, 1)
    w = A.reshape(GH, GL, DIM, DIM).transpose(0, 3, 1, 2).reshape(WIDE, WIDE)
    return _run(k_col, epsilon, w, mu,
                jnp.asarray(_EXP), jnp.asarray(_TILE), jnp.asarray(_FOLD))


# speedup vs baseline: 1.2204x; 1.2204x over previous
import functools

import jax
import jax.numpy as jnp
import numpy as np
from jax.experimental import pallas as pl


BATCH = 16384
NG = 64
DIM = 64
BB = 4096
GC = 8
CHUNK = GC * DIM


GH = 8
GL = 8
WIDE = GH * DIM


def _body(k_ref, eps_ref, w_ref, mu_ref, exp_ref, out_ref):
    f32 = jnp.float32
    eps = eps_ref[...]
    kb = k_ref[...]
    kh = kb >> 3
    kl = kb & 7
    eps_t = jnp.concatenate([eps] * GH, axis=1)
    c1 = jax.lax.broadcasted_iota(jnp.int32, (BB, WIDE), 1)
    e1 = jnp.where((c1 >> 6) == kh, eps_t, 0.0)
    y = jnp.dot(e1, w_ref[...], preferred_element_type=f32)
    i8 = jax.lax.broadcasted_iota(jnp.int32, (BB, GL), 1)
    ohl = (i8 == kl).astype(f32)
    ym = y * jnp.dot(ohl, exp_ref[...], preferred_element_type=f32)
    ym = ym[:, 0:256] + ym[:, 256:512]
    ym = ym[:, 0:128] + ym[:, 128:256]
    ym = ym[:, 0:64] + ym[:, 64:128]
    g64 = jax.lax.broadcasted_iota(jnp.int32, (BB, NG), 1)
    oh = (g64 == kb).astype(f32)
    out_ref[...] = ym + jnp.dot(oh, mu_ref[...], preferred_element_type=f32)


@jax.jit
def _run(k_col, eps, w, mu, exp_m):
    grid = (BATCH // BB,)
    return pl.pallas_call(
        _body,
        grid=grid,
        in_specs=[
            pl.BlockSpec((BB, 1), lambda i: (i, 0)),
            pl.BlockSpec((BB, DIM), lambda i: (i, 0)),
            pl.BlockSpec((WIDE, WIDE), lambda i: (0, 0)),
            pl.BlockSpec((NG, DIM), lambda i: (0, 0)),
            pl.BlockSpec((GH, WIDE), lambda i: (0, 0)),
        ],
        out_specs=pl.BlockSpec((BB, DIM), lambda i: (i, 0)),
        out_shape=jax.ShapeDtypeStruct((BATCH, DIM), jnp.float32),
    )(k_col, eps, w, mu, exp_m)


_EXP = (np.arange(GH)[:, None] == (np.arange(WIDE)[None, :] >> 6)
        ).astype(np.float32)


def kernel(batch_size, k, epsilon, mu, A):
    k_col = k.astype(jnp.int32).reshape(BATCH, 1)
    w = A.reshape(GH, GL, DIM, DIM).transpose(0, 3, 1, 2).reshape(WIDE, WIDE)
    return _run(k_col, epsilon, w, mu, jnp.asarray(_EXP))
